# Initial kernel scaffold; baseline (speedup 1.0000x reference)
import jax
import jax.numpy as jnp
from jax.experimental import pallas as pl


def kernel(x_a, edge_index_a2a, edge_attr_a2a, W_g, b_g, W_f, b_f, W_c, b_c):
    raise NotImplementedError("write your pallas kernel here")


# speedup vs baseline: 2.9870x; 2.9870x over previous
import functools

import jax
import jax.numpy as jnp
from jax import lax
from jax.experimental import pallas as pl
from jax.experimental.pallas import tpu as pltpu
from jax.experimental.pallas import tpu_sc as plsc

N, E, C, DE = 10000, 320000, 128, 4
AW = 16
NC, NS = 2, 16
NW = NC * NS
K = 128
NCHUNK = E // K
FULL = NCHUNK // NW
NP = 10240
RPT = NP // NS
BF = jnp.bfloat16


def _gather_body(x_hbm, src_hbm, dst_hbm, zd_out, zs_out,
                 src_v, dst_v, rows_v, sem):
    cid = lax.axis_index("c")
    sid = lax.axis_index("s")
    wid = sid * NC + cid

    @pl.loop(0, FULL + 1)
    def _chunk(i):
        c = wid + NW * i

        @pl.when(c < NCHUNK)
        def _():
            base = c * K
            pltpu.sync_copy(dst_hbm.at[pl.ds(base, K)], dst_v)
            pltpu.async_copy(x_hbm.at[dst_v], rows_v, sem).wait()
            pltpu.sync_copy(rows_v, zd_out.at[pl.ds(base, K)])
            pltpu.sync_copy(src_hbm.at[pl.ds(base, K)], src_v)
            pltpu.async_copy(x_hbm.at[src_v], rows_v, sem).wait()
            pltpu.sync_copy(rows_v, zs_out.at[pl.ds(base, K)])


def _scatter_body(msg_hbm, dst_hbm, s_out, dst_v, rows_v, s_sh, sem):
    cid = lax.axis_index("c")
    sid = lax.axis_index("s")
    wid = sid * NC + cid
    z16 = jnp.zeros((16,), jnp.float32)

    @pl.loop(0, K)
    def _zero(r):
        for j in range(C // 16):
            rows_v[r, pl.ds(j * 16, 16)] = z16

    for k in range(RPT // K):
        pltpu.sync_copy(rows_v, s_sh.at[pl.ds(sid * RPT + k * K, K)])
    plsc.subcore_barrier()

    @pl.loop(0, FULL + 1)
    def _chunk(i):
        c = wid + NW * i

        @pl.when(c < NCHUNK)
        def _():
            base = c * K
            pltpu.sync_copy(dst_hbm.at[pl.ds(base, K)], dst_v)
            pltpu.sync_copy(msg_hbm.at[pl.ds(base, K)], rows_v)
            pltpu.sync_copy(rows_v, s_sh.at[dst_v], add=True)

    plsc.subcore_barrier()
    pltpu.sync_copy(s_sh.at[pl.ds(sid * RPT, RPT)],
                    s_out.at[pl.ds(cid * NP + sid * RPT, RPT)])


def _count_body(dst_hbm, ea_hbm, a_out, dst_v, ea_v, rows_v, a_sh, sem):
    cid = lax.axis_index("c")
    sid = lax.axis_index("s")
    wid = sid * NC + cid
    z16 = jnp.zeros((16,), jnp.float32)

    @pl.loop(0, K)
    def _zero(r):
        for j in range(C // 16):
            rows_v[r, pl.ds(j * 16, 16)] = z16

    for k in range(RPT // K):
        pltpu.sync_copy(rows_v, a_sh.at[pl.ds(sid * RPT + k * K, K)])
    plsc.subcore_barrier()

    @pl.loop(0, FULL + 1)
    def _chunk(i):
        c = wid + NW * i

        @pl.when(c < NCHUNK)
        def _():
            base = c * K
            pltpu.sync_copy(dst_hbm.at[pl.ds(base, K)], dst_v)
            pltpu.sync_copy(ea_hbm.at[pl.ds(base, K)], ea_v)

            @pl.loop(0, K)
            def _widen(r):
                rows_v[r, pl.ds(0, AW)] = ea_v[r, :]

            pltpu.sync_copy(rows_v, a_sh.at[dst_v], add=True)

    plsc.subcore_barrier()
    pltpu.sync_copy(a_sh.at[pl.ds(sid * RPT, RPT)],
                    a_out.at[pl.ds(cid * NP + sid * RPT, RPT)])


@functools.cache
def _sc_kernels():
    mesh = plsc.VectorSubcoreMesh(core_axis_name="c", subcore_axis_name="s",
                                  num_cores=NC, num_subcores=NS)
    gather_kernel = pl.kernel(
        _gather_body,
        out_type=(
            jax.ShapeDtypeStruct((E, C), jnp.float32),
            jax.ShapeDtypeStruct((E, C), jnp.float32),
        ),
        mesh=mesh,
        scratch_types=(
            pltpu.VMEM((K,), jnp.int32),
            pltpu.VMEM((K,), jnp.int32),
            pltpu.VMEM((K, C), jnp.float32),
            pltpu.SemaphoreType.DMA,
        ),
    )
    scatter_kernel = pl.kernel(
        _scatter_body,
        out_type=jax.ShapeDtypeStruct((NC * NP, C), jnp.float32),
        mesh=mesh,
        scratch_types=(
            pltpu.VMEM((K,), jnp.int32),
            pltpu.VMEM((K, C), jnp.float32),
            pltpu.VMEM_SHARED((NP, C), jnp.float32),
            pltpu.SemaphoreType.DMA,
        ),
    )
    count_kernel = pl.kernel(
        _count_body,
        out_type=jax.ShapeDtypeStruct((NC * NP, C), jnp.float32),
        mesh=mesh,
        scratch_types=(
            pltpu.VMEM((K,), jnp.int32),
            pltpu.VMEM((K, AW), jnp.float32),
            pltpu.VMEM((K, C), jnp.float32),
            pltpu.VMEM_SHARED((NP, C), jnp.float32),
            pltpu.SemaphoreType.DMA,
        ),
    )
    return gather_kernel, scatter_kernel, count_kernel


def _msg_body(zd_ref, zs_ref, ea_ref, wg256_ref, wg8_ref, bg_ref, msg_ref):
    zpair = jnp.concatenate(
        [zd_ref[...].astype(BF), zs_ref[...].astype(BF)], axis=1)
    m256 = jnp.dot(zpair, wg256_ref[...], preferred_element_type=jnp.float32)
    m8 = jnp.dot(ea_ref[...].astype(BF), wg8_ref[...],
                 preferred_element_type=jnp.float32)
    msg_ref[...] = (m256 + m8) + bg_ref[...]


def _msg_kernel(zd, zs, ea8, wg256, wg8, bg2):
    B = 2000
    return pl.pallas_call(
        _msg_body,
        grid=(E // B,),
        in_specs=[
            pl.BlockSpec((B, C), lambda i: (i, 0)),
            pl.BlockSpec((B, C), lambda i: (i, 0)),
            pl.BlockSpec((B, 8), lambda i: (i, 0)),
            pl.BlockSpec((2 * C, C), lambda i: (0, 0)),
            pl.BlockSpec((8, C), lambda i: (0, 0)),
            pl.BlockSpec((1, C), lambda i: (0, 0)),
        ],
        out_specs=pl.BlockSpec((B, C), lambda i: (i, 0)),
        out_shape=jax.ShapeDtypeStruct((E, C), jnp.float32),
    )(zd, zs, ea8, wg256, wg8, bg2)


def _final_body(x_ref, sp_ref, ap_ref, wf_ref, wc_ref, bf_ref, bc_ref,
                y_ref, probs_ref):
    summed = sp_ref[0] + sp_ref[1]
    cnt = ap_ref[0][:, DE:DE + 1] + ap_ref[1][:, DE:DE + 1]
    aggr = summed / jnp.clip(cnt, 1.0, None)
    hin = jnp.concatenate(
        [x_ref[...].astype(BF), aggr.astype(BF)], axis=1)
    h = jnp.dot(hin, wf_ref[...], preferred_element_type=jnp.float32) \
        + bf_ref[...]
    logit = jnp.dot(h.astype(BF), wc_ref[...],
                    preferred_element_type=jnp.float32) + bc_ref[...]
    probs = jax.nn.sigmoid(logit)
    probs_ref[...] = probs
    y_ref[...] = probs > 0.5


def _final_kernel(x_a, sp, ap, wfb, wcb, bf2, bc2):
    B = 2000
    return pl.pallas_call(
        _final_body,
        grid=(N // B,),
        in_specs=[
            pl.BlockSpec((B, C), lambda i: (i, 0)),
            pl.BlockSpec((NC, B, C), lambda i: (0, i, 0)),
            pl.BlockSpec((NC, B, C), lambda i: (0, i, 0)),
            pl.BlockSpec((2 * C, C), lambda i: (0, 0)),
            pl.BlockSpec((C, 1), lambda i: (0, 0)),
            pl.BlockSpec((1, C), lambda i: (0, 0)),
            pl.BlockSpec((1, 1), lambda i: (0, 0)),
        ],
        out_specs=[
            pl.BlockSpec((B, 1), lambda i: (i, 0)),
            pl.BlockSpec((B, 1), lambda i: (i, 0)),
        ],
        out_shape=[
            jax.ShapeDtypeStruct((N, 1), jnp.bool_),
            jax.ShapeDtypeStruct((N, 1), jnp.float32),
        ],
    )(x_a, sp, ap, wfb, wcb, bf2, bc2)


def kernel(x_a, edge_index_a2a, edge_attr_a2a, W_g, b_g, W_f, b_f, W_c, b_c):
    src = edge_index_a2a[0]
    dst = edge_index_a2a[1]
    ea8 = jnp.concatenate(
        [edge_attr_a2a, jnp.zeros((E, 8 - DE), jnp.float32)], axis=1)
    ea16 = jnp.concatenate(
        [edge_attr_a2a,
         jnp.ones((E, 1), jnp.float32),
         jnp.zeros((E, AW - DE - 1), jnp.float32)], axis=1)

    gather_kernel, scatter_kernel, count_kernel = _sc_kernels()
    zd, zs = gather_kernel(x_a, src, dst)

    wg256 = W_g[:2 * C].astype(BF)
    wg8 = jnp.concatenate(
        [W_g[2 * C:], jnp.zeros((8 - DE, C), jnp.float32)], axis=0).astype(BF)
    msg = _msg_kernel(zd, zs, ea8, wg256, wg8, b_g[None, :])

    s_flat = scatter_kernel(msg, dst)
    a_flat = count_kernel(dst, ea16)
    sp = s_flat.reshape(NC, NP, C)
    ap = a_flat.reshape(NC, NP, C)

    y, probs = _final_kernel(x_a, sp, ap, W_f.astype(BF), W_c.astype(BF),
                             b_f[None, :], b_c[None, :])
    return (y, probs)


# speedup vs baseline: 3.2955x; 1.1033x over previous
import functools

import jax
import jax.numpy as jnp
from jax import lax
from jax.experimental import pallas as pl
from jax.experimental.pallas import tpu as pltpu
from jax.experimental.pallas import tpu_sc as plsc

N, E, C, DE = 10000, 320000, 128, 4
AW = 16
NC, NS = 2, 16
NW = NC * NS
K = 128
NCHUNK = E // K
FULL = NCHUNK // NW
NP = 10240
RPT = NP // NS
BF = jnp.bfloat16


def _gather_body(x_hbm, src_hbm, dst_hbm, zd_out, zs_out,
                 src_v, dst_v, rows_v, rows2_v, sem):
    cid = lax.axis_index("c")
    sid = lax.axis_index("s")
    wid = sid * NC + cid

    @pl.loop(0, FULL + 1)
    def _chunk(i):
        c = wid + NW * i

        @pl.when(c < NCHUNK)
        def _():
            base = c * K
            pltpu.sync_copy(dst_hbm.at[pl.ds(base, K)], dst_v)
            pltpu.sync_copy(src_hbm.at[pl.ds(base, K)], src_v)
            d1 = pltpu.async_copy(x_hbm.at[dst_v], rows_v, sem)
            d2 = pltpu.async_copy(x_hbm.at[src_v], rows2_v, sem)
            d1.wait()
            pltpu.sync_copy(rows_v, zd_out.at[pl.ds(base, K)])
            d2.wait()
            pltpu.sync_copy(rows2_v, zs_out.at[pl.ds(base, K)])


def _scatter_body(msg_hbm, dst_hbm, s_out, dst_v, rows_v, s_sh, sem):
    cid = lax.axis_index("c")
    sid = lax.axis_index("s")
    wid = sid * NC + cid
    z16 = jnp.zeros((16,), jnp.float32)

    @pl.loop(0, K)
    def _zero(r):
        for j in range(C // 16):
            rows_v[r, pl.ds(j * 16, 16)] = z16

    for k in range(RPT // K):
        pltpu.sync_copy(rows_v, s_sh.at[pl.ds(sid * RPT + k * K, K)])
    plsc.subcore_barrier()

    @pl.loop(0, FULL + 1)
    def _chunk(i):
        c = wid + NW * i

        @pl.when(c < NCHUNK)
        def _():
            base = c * K
            pltpu.sync_copy(dst_hbm.at[pl.ds(base, K)], dst_v)
            pltpu.sync_copy(msg_hbm.at[pl.ds(base, K)], rows_v)
            pltpu.sync_copy(rows_v, s_sh.at[dst_v], add=True)

    plsc.subcore_barrier()
    pltpu.sync_copy(s_sh.at[pl.ds(sid * RPT, RPT)],
                    s_out.at[pl.ds(cid * NP + sid * RPT, RPT)])


def _count_body(dst_hbm, ea_hbm, a_out, dst_v, ea_v, rows_v, a_sh, sem):
    cid = lax.axis_index("c")
    sid = lax.axis_index("s")
    wid = sid * NC + cid
    z16 = jnp.zeros((16,), jnp.float32)

    @pl.loop(0, K)
    def _zero(r):
        for j in range(C // 16):
            rows_v[r, pl.ds(j * 16, 16)] = z16

    for k in range(RPT // K):
        pltpu.sync_copy(rows_v, a_sh.at[pl.ds(sid * RPT + k * K, K)])
    plsc.subcore_barrier()

    @pl.loop(0, FULL + 1)
    def _chunk(i):
        c = wid + NW * i

        @pl.when(c < NCHUNK)
        def _():
            base = c * K
            pltpu.sync_copy(dst_hbm.at[pl.ds(base, K)], dst_v)
            pltpu.sync_copy(ea_hbm.at[pl.ds(base, K)], ea_v)

            @pl.loop(0, K)
            def _widen(r):
                rows_v[r, pl.ds(0, AW)] = ea_v[r, :]

            pltpu.sync_copy(rows_v, a_sh.at[dst_v], add=True)

    plsc.subcore_barrier()
    pltpu.sync_copy(a_sh.at[pl.ds(sid * RPT, RPT)],
                    a_out.at[pl.ds(cid * NP + sid * RPT, RPT)])


@functools.cache
def _sc_kernels():
    mesh = plsc.VectorSubcoreMesh(core_axis_name="c", subcore_axis_name="s",
                                  num_cores=NC, num_subcores=NS)
    gather_kernel = pl.kernel(
        _gather_body,
        out_type=(
            jax.ShapeDtypeStruct((E, C), jnp.float32),
            jax.ShapeDtypeStruct((E, C), jnp.float32),
        ),
        mesh=mesh,
        scratch_types=(
            pltpu.VMEM((K,), jnp.int32),
            pltpu.VMEM((K,), jnp.int32),
            pltpu.VMEM((K, C), jnp.float32),
            pltpu.VMEM((K, C), jnp.float32),
            pltpu.SemaphoreType.DMA,
        ),
    )
    scatter_kernel = pl.kernel(
        _scatter_body,
        out_type=jax.ShapeDtypeStruct((NC * NP, C), jnp.float32),
        mesh=mesh,
        scratch_types=(
            pltpu.VMEM((K,), jnp.int32),
            pltpu.VMEM((K, C), jnp.float32),
            pltpu.VMEM_SHARED((NP, C), jnp.float32),
            pltpu.SemaphoreType.DMA,
        ),
    )
    count_kernel = pl.kernel(
        _count_body,
        out_type=jax.ShapeDtypeStruct((NC * NP, C), jnp.float32),
        mesh=mesh,
        scratch_types=(
            pltpu.VMEM((K,), jnp.int32),
            pltpu.VMEM((K, AW), jnp.float32),
            pltpu.VMEM((K, C), jnp.float32),
            pltpu.VMEM_SHARED((NP, C), jnp.float32),
            pltpu.SemaphoreType.DMA,
        ),
    )
    return gather_kernel, scatter_kernel, count_kernel


def _msg_body(zd_ref, zs_ref, ea_ref, wg256_ref, wg8_ref, bg_ref, msg_ref):
    zpair = jnp.concatenate(
        [zd_ref[...].astype(BF), zs_ref[...].astype(BF)], axis=1)
    m256 = jnp.dot(zpair, wg256_ref[...], preferred_element_type=jnp.float32)
    m8 = jnp.dot(ea_ref[...].astype(BF), wg8_ref[...],
                 preferred_element_type=jnp.float32)
    msg_ref[...] = (m256 + m8) + bg_ref[...]


def _msg_kernel(zd, zs, ea8, wg256, wg8, bg2):
    B = 2000
    return pl.pallas_call(
        _msg_body,
        grid=(E // B,),
        in_specs=[
            pl.BlockSpec((B, C), lambda i: (i, 0)),
            pl.BlockSpec((B, C), lambda i: (i, 0)),
            pl.BlockSpec((B, 8), lambda i: (i, 0)),
            pl.BlockSpec((2 * C, C), lambda i: (0, 0)),
            pl.BlockSpec((8, C), lambda i: (0, 0)),
            pl.BlockSpec((1, C), lambda i: (0, 0)),
        ],
        out_specs=pl.BlockSpec((B, C), lambda i: (i, 0)),
        out_shape=jax.ShapeDtypeStruct((E, C), jnp.float32),
    )(zd, zs, ea8, wg256, wg8, bg2)


def _final_body(x_ref, sp_ref, ap_ref, wf_ref, wc_ref, bf_ref, bc_ref,
                y_ref, probs_ref):
    summed = sp_ref[0] + sp_ref[1]
    cnt = ap_ref[0][:, DE:DE + 1] + ap_ref[1][:, DE:DE + 1]
    aggr = summed / jnp.clip(cnt, 1.0, None)
    hin = jnp.concatenate(
        [x_ref[...].astype(BF), aggr.astype(BF)], axis=1)
    h = jnp.dot(hin, wf_ref[...], preferred_element_type=jnp.float32) \
        + bf_ref[...]
    logit = jnp.dot(h.astype(BF), wc_ref[...],
                    preferred_element_type=jnp.float32) + bc_ref[...]
    probs = jax.nn.sigmoid(logit)
    probs_ref[...] = probs
    y_ref[...] = probs > 0.5


def _final_kernel(x_a, sp, ap, wfb, wcb, bf2, bc2):
    B = 2000
    return pl.pallas_call(
        _final_body,
        grid=(N // B,),
        in_specs=[
            pl.BlockSpec((B, C), lambda i: (i, 0)),
            pl.BlockSpec((NC, B, C), lambda i: (0, i, 0)),
            pl.BlockSpec((NC, B, C), lambda i: (0, i, 0)),
            pl.BlockSpec((2 * C, C), lambda i: (0, 0)),
            pl.BlockSpec((C, 1), lambda i: (0, 0)),
            pl.BlockSpec((1, C), lambda i: (0, 0)),
            pl.BlockSpec((1, 1), lambda i: (0, 0)),
        ],
        out_specs=[
            pl.BlockSpec((B, 1), lambda i: (i, 0)),
            pl.BlockSpec((B, 1), lambda i: (i, 0)),
        ],
        out_shape=[
            jax.ShapeDtypeStruct((N, 1), jnp.bool_),
            jax.ShapeDtypeStruct((N, 1), jnp.float32),
        ],
    )(x_a, sp, ap, wfb, wcb, bf2, bc2)


def kernel(x_a, edge_index_a2a, edge_attr_a2a, W_g, b_g, W_f, b_f, W_c, b_c):
    src = edge_index_a2a[0]
    dst = edge_index_a2a[1]
    ea8 = jnp.concatenate(
        [edge_attr_a2a, jnp.zeros((E, 8 - DE), jnp.float32)], axis=1)
    ea16 = jnp.concatenate(
        [edge_attr_a2a,
         jnp.ones((E, 1), jnp.float32),
         jnp.zeros((E, AW - DE - 1), jnp.float32)], axis=1)

    gather_kernel, scatter_kernel, count_kernel = _sc_kernels()
    zd, zs = gather_kernel(x_a, src, dst)

    wg256 = W_g[:2 * C].astype(BF)
    wg8 = jnp.concatenate(
        [W_g[2 * C:], jnp.zeros((8 - DE, C), jnp.float32)], axis=0).astype(BF)
    msg = _msg_kernel(zd, zs, ea8, wg256, wg8, b_g[None, :])

    s_flat = scatter_kernel(msg, dst)
    a_flat = count_kernel(dst, ea16)
    sp = s_flat.reshape(NC, NP, C)
    ap = a_flat.reshape(NC, NP, C)

    y, probs = _final_kernel(x_a, sp, ap, W_f.astype(BF), W_c.astype(BF),
                             b_f[None, :], b_c[None, :])
    return (y, probs)
